# baseline (speedup 1.0000x reference)
import functools

import jax
import jax.numpy as jnp
from jax.experimental import pallas as pl
from jax.experimental.pallas import tpu as pltpu


def _se_kernel(x_ref, w_ref, o_ref, *, inv_hw, c_red):
    x = x_ref[...]
    pooled = jnp.sum(x, axis=0) * inv_hw
    w1 = w_ref[:c_red, :]
    w2t = w_ref[c_red:, :]
    h = jax.lax.dot_general(
        pooled, w1, (((1,), (1,)), ((), ())),
        preferred_element_type=jnp.float32)
    h = jnp.maximum(h, 0.0)
    g = jax.lax.dot_general(
        h, w2t, (((1,), (0,)), ((), ())),
        preferred_element_type=jnp.float32)
    g = jax.nn.sigmoid(g)
    o_ref[...] = x * g[None, :, :]


def kernel(x, w1, w2):
    b, c, h, w = x.shape
    hw = h * w
    c_red = w1.shape[0]
    itemsize = jnp.dtype(x.dtype).itemsize

    xt = jnp.transpose(x.reshape(b, c, hw), (2, 0, 1))
    wcat = jnp.concatenate([w1, w2.T], axis=0)

    tb = 8
    while b % tb:
        tb -= 1

    w_bytes = int((w1.size + w2.size) * jnp.dtype(w1.dtype).itemsize)
    cost = pl.CostEstimate(
        flops=int(2 * b * c * hw + 4 * b * c * c_red),
        transcendentals=int(b * c),
        bytes_accessed=int(2 * b * c * hw * itemsize + w_bytes))

    out_t = pl.pallas_call(
        functools.partial(_se_kernel, inv_hw=1.0 / hw, c_red=c_red),
        out_shape=jax.ShapeDtypeStruct((hw, b, c), x.dtype),
        grid=(b // tb,),
        in_specs=[
            pl.BlockSpec((hw, tb, c), lambda i: (0, i, 0)),
            pl.BlockSpec(wcat.shape, lambda i: (0, 0)),
        ],
        out_specs=pl.BlockSpec((hw, tb, c), lambda i: (0, i, 0)),
        compiler_params=pltpu.CompilerParams(
            dimension_semantics=("parallel",),
            vmem_limit_bytes=48 * 1024 * 1024),
        cost_estimate=cost,
    )(xt, wcat)

    return jnp.transpose(out_t, (1, 2, 0)).reshape(b, c, h, w)


# speedup vs baseline: 1.0060x; 1.0060x over previous
import functools

import jax
import jax.numpy as jnp
from jax.experimental import pallas as pl
from jax.experimental.pallas import tpu as pltpu


def _se_kernel(x_ref, w_ref, o_ref, *, c_red):
    x = x_ref[...]
    pooled = jnp.sum(x, axis=0)
    w1 = w_ref[:c_red, :]
    w2t = w_ref[c_red:, :]
    h = jax.lax.dot_general(
        pooled, w1, (((1,), (1,)), ((), ())),
        preferred_element_type=jnp.float32)
    h = jnp.maximum(h, 0.0)
    g = jax.lax.dot_general(
        h, w2t, (((1,), (0,)), ((), ())),
        preferred_element_type=jnp.float32)
    g = jax.nn.sigmoid(g)
    o_ref[...] = x * g[None, :, :]


def kernel(x, w1, w2):
    b, c, h, w = x.shape
    hw = h * w
    c_red = w1.shape[0]
    itemsize = jnp.dtype(x.dtype).itemsize

    xt = jnp.transpose(x.reshape(b, c, hw), (2, 0, 1))
    wcat = jnp.concatenate([w1 * (1.0 / hw), w2.T], axis=0)

    tb = 16
    while b % tb:
        tb -= 1

    w_bytes = int((w1.size + w2.size) * jnp.dtype(w1.dtype).itemsize)
    cost = pl.CostEstimate(
        flops=int(2 * b * c * hw + 4 * b * c * c_red),
        transcendentals=int(b * c),
        bytes_accessed=int(2 * b * c * hw * itemsize + w_bytes))

    out_t = pl.pallas_call(
        functools.partial(_se_kernel, c_red=c_red),
        out_shape=jax.ShapeDtypeStruct((hw, b, c), x.dtype),
        grid=(b // tb,),
        in_specs=[
            pl.BlockSpec((hw, tb, c), lambda i: (0, i, 0)),
            pl.BlockSpec(wcat.shape, lambda i: (0, 0)),
        ],
        out_specs=pl.BlockSpec((hw, tb, c), lambda i: (0, i, 0)),
        compiler_params=pltpu.CompilerParams(
            dimension_semantics=("parallel",),
            vmem_limit_bytes=48 * 1024 * 1024),
        cost_estimate=cost,
    )(xt, wcat)

    return jnp.transpose(out_t, (1, 2, 0)).reshape(b, c, h, w)


# speedup vs baseline: 1.0596x; 1.0533x over previous
import functools

import jax
import jax.numpy as jnp
from jax.experimental import pallas as pl
from jax.experimental.pallas import tpu as pltpu


def _se_kernel(x_ref, w_ref, o_ref, *, inv_hw, c_red):
    x = x_ref[...]
    pooled = jnp.sum(x, axis=0) * inv_hw
    w1 = w_ref[:c_red, :]
    w2t = w_ref[c_red:, :]
    h = jax.lax.dot_general(
        pooled, w1, (((1,), (1,)), ((), ())),
        preferred_element_type=jnp.float32)
    h = jnp.maximum(h, 0.0)
    g = jax.lax.dot_general(
        h, w2t, (((1,), (0,)), ((), ())),
        preferred_element_type=jnp.float32)
    g = jax.nn.sigmoid(g)
    o_ref[...] = x * g[None, :, :]


def kernel(x, w1, w2):
    b, c, h, w = x.shape
    hw = h * w
    c_red = w1.shape[0]
    itemsize = jnp.dtype(x.dtype).itemsize

    xt = jnp.transpose(x.reshape(b, c, hw), (2, 0, 1))
    wcat = jnp.concatenate([w1, w2.T], axis=0)

    tb = 16
    while b % tb:
        tb -= 1

    w_bytes = int((w1.size + w2.size) * jnp.dtype(w1.dtype).itemsize)
    cost = pl.CostEstimate(
        flops=int(2 * b * c * hw + 4 * b * c * c_red),
        transcendentals=int(b * c),
        bytes_accessed=int(2 * b * c * hw * itemsize + w_bytes))

    out_t = pl.pallas_call(
        functools.partial(_se_kernel, inv_hw=1.0 / hw, c_red=c_red),
        out_shape=jax.ShapeDtypeStruct((hw, b, c), x.dtype),
        grid=(b // tb,),
        in_specs=[
            pl.BlockSpec((hw, tb, c), lambda i: (0, i, 0)),
            pl.BlockSpec(wcat.shape, lambda i: (0, 0)),
        ],
        out_specs=pl.BlockSpec((hw, tb, c), lambda i: (0, i, 0)),
        compiler_params=pltpu.CompilerParams(
            dimension_semantics=("parallel",),
            vmem_limit_bytes=48 * 1024 * 1024),
        cost_estimate=cost,
    )(xt, wcat)

    return jnp.transpose(out_t, (1, 2, 0)).reshape(b, c, h, w)
